# baseline (speedup 1.0000x reference)
import functools

import jax
import jax.numpy as jnp
from jax import lax
from jax.experimental import pallas as pl
from jax.experimental.pallas import tpu as pltpu
from jax.experimental.pallas import tpu_sc as plsc

_BATCH = 4096
_SEQ = 50
_DIM = 128
_NC, _NS = 2, 16
_NW = _NC * _NS
_RPW = _BATCH // _NW
_NB = 4

_mesh = plsc.VectorSubcoreMesh(core_axis_name="c", subcore_axis_name="s")


@functools.partial(
    pl.kernel,
    out_type=jax.ShapeDtypeStruct((_BATCH, _SEQ, _DIM), jnp.float32),
    mesh=_mesh,
    scratch_types=[
        pltpu.VMEM((_RPW, _SEQ), jnp.int32),
        pltpu.VMEM((_NB, _SEQ, _DIM), jnp.float32),
        [pltpu.SemaphoreType.DMA] * _NB,
        [pltpu.SemaphoreType.DMA] * _NB,
    ],
    compiler_params=pltpu.CompilerParams(use_tc_tiling_on_sc=False),
)
def _embed_gather(idx_hbm, table_hbm, out_hbm, idx_v, bufs, gsems, wsems):
    wid = lax.axis_index("s") * _NC + lax.axis_index("c")
    base = wid * _RPW
    pltpu.sync_copy(idx_hbm.at[pl.ds(base, _RPW)], idx_v)

    def fire_gather(r, b):
        pltpu.async_copy(table_hbm.at[idx_v.at[r]], bufs.at[b], gsems[b])

    def wait_gather(b):
        pltpu.make_async_copy(
            table_hbm.at[pl.ds(0, _SEQ)], bufs.at[b], gsems[b]
        ).wait()

    def fire_write(r, b):
        pltpu.async_copy(bufs.at[b], out_hbm.at[base + r], wsems[b])

    def wait_write(b):
        pltpu.make_async_copy(bufs.at[b], out_hbm.at[0], wsems[b]).wait()

    for b in range(_NB):
        fire_gather(b, b)

    @pl.loop(0, _RPW - _NB, step=_NB)
    def _round(r0):
        for b in range(_NB):
            wait_gather(b)
            fire_write(r0 + b, b)
        for b in range(_NB):
            wait_write(b)
            fire_gather(r0 + _NB + b, b)

    for b in range(_NB):
        wait_gather(b)
        fire_write(_RPW - _NB + b, b)
    for b in range(_NB):
        wait_write(b)


def kernel(token_ids, weight):
    return _embed_gather(token_ids, weight)


# speedup vs baseline: 1.7548x; 1.7548x over previous
import functools

import jax
import jax.numpy as jnp
from jax import lax
from jax.experimental import pallas as pl
from jax.experimental.pallas import tpu as pltpu
from jax.experimental.pallas import tpu_sc as plsc

_BATCH = 4096
_SEQ = 50
_DIM = 128
_NC, _NS = 2, 16
_NW = _NC * _NS
_RPW = _BATCH // _NW
_NB = 4

_mesh = plsc.VectorSubcoreMesh(core_axis_name="c", subcore_axis_name="s")


@functools.partial(
    pl.kernel,
    out_type=jax.ShapeDtypeStruct((_BATCH, _SEQ, _DIM), jnp.float32),
    mesh=_mesh,
    scratch_types=[
        pltpu.VMEM((_RPW, _SEQ), jnp.int32),
        pltpu.VMEM((_NB, _SEQ, _DIM), jnp.float32),
        [pltpu.SemaphoreType.DMA] * _NB,
        [pltpu.SemaphoreType.DMA] * _NB,
    ],
)
def _embed_gather(idx_hbm, table_hbm, out_hbm, idx_v, bufs, gsems, wsems):
    wid = lax.axis_index("s") * _NC + lax.axis_index("c")
    base = wid * _RPW
    pltpu.sync_copy(idx_hbm.at[pl.ds(base, _RPW)], idx_v)

    def fire_gather(r, b):
        pltpu.async_copy(table_hbm.at[idx_v.at[r]], bufs.at[b], gsems[b])

    def wait_gather(b):
        pltpu.make_async_copy(out_hbm.at[0], bufs.at[b], gsems[b]).wait()

    def fire_write(r, b):
        pltpu.async_copy(bufs.at[b], out_hbm.at[base + r], wsems[b])

    def wait_write(b):
        pltpu.make_async_copy(bufs.at[b], out_hbm.at[0], wsems[b]).wait()

    for b in range(_NB):
        fire_gather(b, b)

    @pl.loop(0, _RPW - _NB, step=_NB)
    def _round(r0):
        for b in range(_NB):
            wait_gather(b)
            fire_write(r0 + b, b)
        for b in range(_NB):
            wait_write(b)
            fire_gather(r0 + _NB + b, b)

    for b in range(_NB):
        wait_gather(b)
        fire_write(_RPW - _NB + b, b)
    for b in range(_NB):
        wait_write(b)


def kernel(token_ids, weight):
    return _embed_gather(token_ids, weight)


# speedup vs baseline: 3.1727x; 1.8080x over previous
import functools

import jax
import jax.numpy as jnp
from jax import lax
from jax.experimental import pallas as pl
from jax.experimental.pallas import tpu as pltpu
from jax.experimental.pallas import tpu_sc as plsc

_BATCH = 4096
_SEQ = 50
_DIM = 128
_NC, _NS = 2, 16
_NW = _NC * _NS
_CPW = _BATCH // _NW
_NB = 5

_mesh = plsc.VectorSubcoreMesh(core_axis_name="c", subcore_axis_name="s")


@functools.partial(
    pl.kernel,
    out_type=jax.ShapeDtypeStruct((_SEQ, _BATCH, _DIM), jnp.float32),
    mesh=_mesh,
    scratch_types=[
        pltpu.VMEM((_SEQ, _CPW), jnp.int32),
        pltpu.VMEM((_NB, _CPW, _DIM), jnp.float32),
        [pltpu.SemaphoreType.DMA] * _NB,
        [pltpu.SemaphoreType.DMA] * _NB,
    ],
)
def _embed_gather(idx_hbm, table_hbm, out_hbm, idx_v, bufs, gsems, wsems):
    wid = lax.axis_index("s") * _NC + lax.axis_index("c")
    c0 = wid * _CPW
    pltpu.sync_copy(idx_hbm.at[:, pl.ds(c0, _CPW)], idx_v)

    def fire_gather(t, b):
        pltpu.async_copy(table_hbm.at[idx_v.at[t]], bufs.at[b], gsems[b])

    def wait_gather(b):
        pltpu.make_async_copy(
            table_hbm.at[pl.ds(0, _CPW)], bufs.at[b], gsems[b]
        ).wait()

    def fire_write(t, b):
        pltpu.async_copy(
            bufs.at[b], out_hbm.at[t, pl.ds(c0, _CPW)], wsems[b]
        )

    def wait_write(b):
        pltpu.make_async_copy(
            bufs.at[b], out_hbm.at[0, pl.ds(c0, _CPW)], wsems[b]
        ).wait()

    for b in range(_NB):
        fire_gather(b, b)

    @pl.loop(0, _SEQ - _NB, step=_NB)
    def _round(t0):
        for b in range(_NB):
            wait_gather(b)
            fire_write(t0 + b, b)
        for b in range(_NB):
            wait_write(b)
            fire_gather(t0 + _NB + b, b)

    for b in range(_NB):
        wait_gather(b)
        fire_write(_SEQ - _NB + b, b)
    for b in range(_NB):
        wait_write(b)


def kernel(token_ids, weight):
    out = _embed_gather(token_ids.T, weight)
    return jnp.transpose(out, (1, 0, 2))


# speedup vs baseline: 3.1803x; 1.0024x over previous
import functools

import jax
import jax.numpy as jnp
from jax import lax
from jax.experimental import pallas as pl
from jax.experimental.pallas import tpu as pltpu
from jax.experimental.pallas import tpu_sc as plsc

_BATCH = 4096
_SEQ = 50
_DIM = 128
_NC, _NS = 2, 16
_NW = _NC * _NS
_CPW = _BATCH // _NW
_NB = 5

_mesh = plsc.VectorSubcoreMesh(core_axis_name="c", subcore_axis_name="s")


@functools.partial(
    pl.kernel,
    out_type=jax.ShapeDtypeStruct((_SEQ, _BATCH, _DIM), jnp.float32),
    mesh=_mesh,
    scratch_types=[
        pltpu.VMEM((_SEQ, _CPW), jnp.int32),
        pltpu.VMEM((_NB, _CPW, _DIM), jnp.float32),
        [pltpu.SemaphoreType.DMA] * _NB,
        [pltpu.SemaphoreType.DMA] * _NB,
    ],
)
def _embed_gather(idx_hbm, table_hbm, out_hbm, idx_v, bufs, gsems, wsems):
    wid = lax.axis_index("s") * _NC + lax.axis_index("c")
    c0 = wid * _CPW
    pltpu.sync_copy(idx_hbm.at[:, pl.ds(c0, _CPW)], idx_v)

    def fire_gather(t, b):
        pltpu.async_copy(table_hbm.at[idx_v.at[t]], bufs.at[b], gsems[b])

    def wait_gather(b):
        pltpu.make_async_copy(
            table_hbm.at[pl.ds(0, _CPW)], bufs.at[b], gsems[b]
        ).wait()

    def fire_write(t, b):
        pltpu.async_copy(
            bufs.at[b], out_hbm.at[t, pl.ds(c0, _CPW)], wsems[b]
        )

    def wait_write(b):
        pltpu.make_async_copy(
            bufs.at[b], out_hbm.at[0, pl.ds(c0, _CPW)], wsems[b]
        ).wait()

    for b in range(_NB):
        fire_gather(b, b)

    @pl.loop(0, _SEQ - _NB, step=_NB)
    def _round(t0):
        for b in range(_NB):
            wait_gather(b)
            fire_write(t0 + b, b)
        for b in range(_NB):
            wait_write(b)
            fire_gather(t0 + _NB + b, b)

    for b in range(_NB):
        wait_gather(b)
        fire_write(_SEQ - _NB + b, b)
    for b in range(_NB):
        wait_write(b)


def kernel(token_ids, weight):
    idx = token_ids.astype(jnp.int32)
    out = _embed_gather(idx.T, weight)
    return jnp.transpose(out, (1, 0, 2))
